# Initial kernel scaffold; baseline (speedup 1.0000x reference)
import jax
import jax.numpy as jnp
from jax.experimental import pallas as pl


def kernel(x, params, edge_index):
    raise NotImplementedError("write your pallas kernel here")


# speedup vs baseline: 3.1010x; 3.1010x over previous
import functools

import jax
import jax.numpy as jnp
from jax import lax
from jax.experimental import pallas as pl
from jax.experimental.pallas import tpu as pltpu
from jax.experimental.pallas import tpu_sc as plsc

EPS = 1e-5
N_CORES = 2
N_SUBCORES = 16
NW = N_CORES * N_SUBCORES
CHUNK = 128
NEG = -3.0e38


def _lrelu(x):
    return jnp.where(x >= 0, x, 0.2 * x)


def _make_segsum(n_pad, d, n_chunks):
    stripe = n_pad // N_SUBCORES
    n_stage = stripe // CHUNK

    mesh = plsc.VectorSubcoreMesh(core_axis_name="c", subcore_axis_name="s")

    def body(u_hbm, src_hbm, dst_hbm, out_hbm, src_v, dst_v, rows_v, s_sh,
             sem):
        c = lax.axis_index("c")
        s = lax.axis_index("s")
        wid = c * N_SUBCORES + s

        def zrow(i, carry):
            for k in range(d // 16):
                rows_v[i, pl.ds(k * 16, 16)] = jnp.zeros((16,), jnp.float32)
            return carry
        lax.fori_loop(0, CHUNK, zrow, 0)

        for t in range(n_stage):
            r0 = s * stripe + t * CHUNK
            pltpu.sync_copy(rows_v, s_sh.at[pl.ds(r0, CHUNK)])
        plsc.subcore_barrier()

        pltpu.sync_copy(src_hbm.at[pl.ds(wid * n_chunks, n_chunks)], src_v)
        pltpu.sync_copy(dst_hbm.at[pl.ds(wid * n_chunks, n_chunks)], dst_v)

        def step(j, carry):
            pltpu.async_copy(u_hbm.at[src_v.at[j]], rows_v, sem).wait()
            pltpu.sync_copy(rows_v, s_sh.at[dst_v.at[j]], add=True)
            return carry
        lax.fori_loop(0, n_chunks, step, 0)
        plsc.subcore_barrier()

        for t in range(n_stage):
            r0 = s * stripe + t * CHUNK
            pltpu.sync_copy(s_sh.at[pl.ds(r0, CHUNK)], rows_v)
            pltpu.sync_copy(rows_v, out_hbm.at[pl.ds(c * n_pad + r0, CHUNK)])

    return pl.kernel(
        body,
        out_type=[jax.ShapeDtypeStruct((2 * n_pad, d), jnp.float32)],
        mesh=mesh,
        scratch_types=[
            pltpu.VMEM((n_chunks, CHUNK), jnp.int32),
            pltpu.VMEM((n_chunks, CHUNK), jnp.int32),
            pltpu.VMEM((CHUNK, d), jnp.float32),
            pltpu.VMEM_SHARED((n_pad, d), jnp.float32),
            pltpu.SemaphoreType.DMA,
        ])


def _make_deg(n_pad, n_chunks):
    stripe = n_pad // N_SUBCORES
    n_stage = stripe // CHUNK

    mesh = plsc.VectorSubcoreMesh(core_axis_name="c", subcore_axis_name="s")

    def body(dst_hbm, out_hbm, dst_v, ones_v, deg_sh, sem):
        c = lax.axis_index("c")
        s = lax.axis_index("s")
        wid = c * N_SUBCORES + s

        def zrow(i, carry):
            for k in range(128 // 16):
                ones_v[i, pl.ds(k * 16, 16)] = jnp.zeros((16,), jnp.float32)
            return carry
        lax.fori_loop(0, CHUNK, zrow, 0)
        for t in range(n_stage):
            r0 = s * stripe + t * CHUNK
            pltpu.sync_copy(ones_v, deg_sh.at[pl.ds(r0, CHUNK)])

        def orow(i, carry):
            for k in range(128 // 16):
                ones_v[i, pl.ds(k * 16, 16)] = jnp.ones((16,), jnp.float32)
            return carry
        lax.fori_loop(0, CHUNK, orow, 0)
        plsc.subcore_barrier()

        pltpu.sync_copy(dst_hbm.at[pl.ds(wid * n_chunks, n_chunks)], dst_v)

        def step(j, carry):
            pltpu.sync_copy(ones_v, deg_sh.at[dst_v.at[j]], add=True)
            return carry
        lax.fori_loop(0, n_chunks, step, 0)
        plsc.subcore_barrier()

        for t in range(n_stage):
            r0 = s * stripe + t * CHUNK
            pltpu.sync_copy(deg_sh.at[pl.ds(r0, CHUNK)], ones_v)
            pltpu.sync_copy(ones_v, out_hbm.at[pl.ds(c * n_pad + r0, CHUNK)])

    return pl.kernel(
        body,
        out_type=[jax.ShapeDtypeStruct((2 * n_pad, 128), jnp.float32)],
        mesh=mesh,
        scratch_types=[
            pltpu.VMEM((n_chunks, CHUNK), jnp.int32),
            pltpu.VMEM((CHUNK, 128), jnp.float32),
            pltpu.VMEM_SHARED((n_pad, 128), jnp.float32),
            pltpu.SemaphoreType.DMA,
        ])


_BM = 256


def _tc_mlp2(x, w1t, w2t, bias):
    n = x.shape[0]

    def body(x_ref, w1_ref, w2_ref, b_ref, o_ref):
        h = jnp.dot(x_ref[...], w1_ref[...],
                    preferred_element_type=jnp.float32) + b_ref[0, :][None, :]
        h = _lrelu(h)
        h = jnp.dot(h, w2_ref[...],
                    preferred_element_type=jnp.float32) + b_ref[1, :][None, :]
        o_ref[...] = _lrelu(h)

    return pl.pallas_call(
        body,
        grid=(n // _BM,),
        in_specs=[pl.BlockSpec((_BM, 128), lambda i: (i, 0)),
                  pl.BlockSpec((128, 128), lambda i: (0, 0)),
                  pl.BlockSpec((128, 128), lambda i: (0, 0)),
                  pl.BlockSpec((8, 128), lambda i: (0, 0))],
        out_specs=pl.BlockSpec((_BM, 128), lambda i: (i, 0)),
        out_shape=jax.ShapeDtypeStruct((n, 128), jnp.float32),
    )(x, w1t, w2t, bias)


def _tc_mid(s0, s1, d0, d1, wlt, w1t, w2t, bias):
    n = s0.shape[0]

    def body(s0r, s1r, d0r, d1r, wl, w1, w2, br, o_ref):
        ssum = s0r[...] + s1r[...]
        deg = d0r[:, 0:1] + d1r[:, 0:1]
        mean = ssum / jnp.maximum(deg, 1.0)
        h = jnp.dot(mean, wl[...],
                    preferred_element_type=jnp.float32) + br[0, :][None, :]
        h = _lrelu(h)
        h = jnp.dot(h, w1[...],
                    preferred_element_type=jnp.float32) + br[1, :][None, :]
        h = _lrelu(h)
        h = jnp.dot(h, w2[...],
                    preferred_element_type=jnp.float32) + br[2, :][None, :]
        o_ref[...] = _lrelu(h)

    return pl.pallas_call(
        body,
        grid=(n // _BM,),
        in_specs=[pl.BlockSpec((_BM, 128), lambda i: (i, 0)),
                  pl.BlockSpec((_BM, 128), lambda i: (i, 0)),
                  pl.BlockSpec((_BM, 128), lambda i: (i, 0)),
                  pl.BlockSpec((_BM, 128), lambda i: (i, 0)),
                  pl.BlockSpec((128, 128), lambda i: (0, 0)),
                  pl.BlockSpec((128, 128), lambda i: (0, 0)),
                  pl.BlockSpec((128, 128), lambda i: (0, 0)),
                  pl.BlockSpec((8, 128), lambda i: (0, 0))],
        out_specs=pl.BlockSpec((_BM, 128), lambda i: (i, 0)),
        out_shape=jax.ShapeDtypeStruct((n, 128), jnp.float32),
    )(s0, s1, d0, d1, wlt, w1t, w2t, bias)


def _tc_final(s0, s1, d0, d1, wlt, bias, n_real):
    n = s0.shape[0]

    def body(s0r, s1r, d0r, d1r, wl, br, o_ref):
        i = pl.program_id(0)
        ssum = s0r[...] + s1r[...]
        deg = d0r[:, 0:1] + d1r[:, 0:1]
        mean = ssum / jnp.maximum(deg, 1.0)
        h = jnp.dot(mean, wl[...],
                    preferred_element_type=jnp.float32) + br[0, :][None, :]
        rid = i * _BM + lax.broadcasted_iota(jnp.int32, (_BM, 1), 0)
        h = jnp.where(rid < n_real, h, NEG)
        bmax = jnp.broadcast_to(jnp.max(h, axis=0, keepdims=True), (8, 128))

        @pl.when(i == 0)
        def _():
            o_ref[...] = jnp.full((8, 128), NEG, jnp.float32)

        o_ref[...] = jnp.maximum(o_ref[...], bmax)

    return pl.pallas_call(
        body,
        grid=(n // _BM,),
        in_specs=[pl.BlockSpec((_BM, 128), lambda i: (i, 0)),
                  pl.BlockSpec((_BM, 128), lambda i: (i, 0)),
                  pl.BlockSpec((_BM, 128), lambda i: (i, 0)),
                  pl.BlockSpec((_BM, 128), lambda i: (i, 0)),
                  pl.BlockSpec((128, 128), lambda i: (0, 0)),
                  pl.BlockSpec((8, 128), lambda i: (0, 0))],
        out_specs=pl.BlockSpec((8, 128), lambda i: (0, 0)),
        out_shape=jax.ShapeDtypeStruct((8, 128), jnp.float32),
    )(s0, s1, d0, d1, wlt, bias)


def _tc_head(mx, w1t, w2t, w3t, bias, n_out):
    def body(m_ref, w1, w2, w3, br, o_ref):
        h = jnp.dot(m_ref[...], w1[...],
                    preferred_element_type=jnp.float32) + br[0, :][None, :]
        h = _lrelu(h)
        h = jnp.dot(h, w2[...],
                    preferred_element_type=jnp.float32) + br[1, :][None, :]
        h = _lrelu(h)
        z = jnp.dot(h, w3[...],
                    preferred_element_type=jnp.float32) + br[2, :][None, :]
        col = lax.broadcasted_iota(jnp.int32, (1, 128), 1)
        valid = col < n_out
        zm = jnp.where(valid, z, NEG)
        m = jnp.max(zm, axis=1, keepdims=True)
        e = jnp.where(valid, jnp.exp(z - m), 0.0)
        lse = jnp.log(jnp.sum(e, axis=1, keepdims=True))
        o_ref[...] = z - m - lse

    return pl.pallas_call(
        body,
        grid=(1,),
        in_specs=[pl.BlockSpec((8, 128), lambda i: (0, 0)),
                  pl.BlockSpec((128, 128), lambda i: (0, 0)),
                  pl.BlockSpec((128, 128), lambda i: (0, 0)),
                  pl.BlockSpec((128, 128), lambda i: (0, 0)),
                  pl.BlockSpec((8, 128), lambda i: (0, 0))],
        out_specs=pl.BlockSpec((8, 128), lambda i: (0, 0)),
        out_shape=jax.ShapeDtypeStruct((8, 128), jnp.float32),
    )(mx, w1t, w2t, w3t, bias)


def _fold(lin, bn):
    s = bn["gamma"] / jnp.sqrt(1.0 + EPS)
    return (lin["W"] * s[:, None]).T, lin["b"] * s + bn["beta"]


def _bias_table(*rows):
    b = jnp.zeros((8, 128), jnp.float32)
    for i, r in enumerate(rows):
        b = b.at[i, : r.shape[0]].set(r)
    return b


def _sc_segsum(u, src2d, dst2d, n_pad):
    (out,) = _make_segsum(n_pad, u.shape[1], src2d.shape[0] // NW)(
        u, src2d, dst2d)
    return out


def _sc_deg(dst2d, n_pad):
    (out,) = _make_deg(n_pad, dst2d.shape[0] // NW)(dst2d)
    return out


def kernel(x, params, edge_index):
    p = params
    n, d = x.shape
    e = edge_index.shape[1]

    n_pad = ((n + 1 + 2047) // 2048) * 2048
    eq = NW * CHUNK * 8
    e_pad = ((e + eq - 1) // eq) * eq

    xp = jnp.pad(x, ((0, n_pad - n), (0, 0)))
    pad = jnp.full((e_pad - e,), n, jnp.int32)
    src2d = jnp.concatenate([edge_index[0], pad]).reshape(e_pad // CHUNK,
                                                          CHUNK)
    dst2d = jnp.concatenate([edge_index[1], pad]).reshape(e_pad // CHUNK,
                                                          CHUNK)

    w1a, b1a = _fold(p["mp1"]["fc1"], p["mp1"]["bn1"])
    w2a, b2a = _fold(p["mp1"]["fc2"], p["mp1"]["bn2"])
    wl1, bl1 = _fold(p["mp1"]["lin"], p["post1"])
    w1b, b1b = _fold(p["mp2"]["fc1"], p["mp2"]["bn1"])
    w2b, b2b = _fold(p["mp2"]["fc2"], p["mp2"]["bn2"])
    wl2, bl2 = _fold(p["mp2"]["lin"], p["post2"])
    w1c, b1c = _fold(p["mp3"]["fc1"], p["mp3"]["bn1"])
    w2c, b2c = _fold(p["mp3"]["fc2"], p["mp3"]["bn2"])
    wl3 = p["mp3"]["lin"]["W"].T
    bl3 = p["mp3"]["lin"]["b"]
    wf1, bf1 = _fold(p["fc1"]["lin"], p["fc1"]["bn"])
    wf2, bf2 = _fold(p["fc2"]["lin"], p["fc2"]["bn"])
    n_out = p["fc_final"]["W"].shape[0]
    wf3 = jnp.zeros((128, 128), jnp.float32).at[:, :n_out].set(
        p["fc_final"]["W"].T)
    bf3 = p["fc_final"]["b"]

    deg_parts = _sc_deg(dst2d, n_pad)
    d0, d1 = deg_parts[:n_pad], deg_parts[n_pad:]

    u1 = _tc_mlp2(xp, w1a, w2a, _bias_table(b1a, b2a))
    s_parts = _sc_segsum(u1, src2d, dst2d, n_pad)
    u2 = _tc_mid(s_parts[:n_pad], s_parts[n_pad:], d0, d1,
                 wl1, w1b, w2b, _bias_table(bl1, b1b, b2b))
    s_parts = _sc_segsum(u2, src2d, dst2d, n_pad)
    u3 = _tc_mid(s_parts[:n_pad], s_parts[n_pad:], d0, d1,
                 wl2, w1c, w2c, _bias_table(bl2, b1c, b2c))
    s_parts = _sc_segsum(u3, src2d, dst2d, n_pad)
    mx = _tc_final(s_parts[:n_pad], s_parts[n_pad:], d0, d1,
                   wl3, _bias_table(bl3), n)
    out = _tc_head(mx, wf1, wf2, wf3, _bias_table(bf1, bf2, bf3), n_out)
    return out[0:1, 0:n_out]


# speedup vs baseline: 3.5047x; 1.1302x over previous
import functools

import jax
import jax.numpy as jnp
from jax import lax
from jax.experimental import pallas as pl
from jax.experimental.pallas import tpu as pltpu
from jax.experimental.pallas import tpu_sc as plsc

EPS = 1e-5
N_CORES = 2
N_SUBCORES = 16
NW = N_CORES * N_SUBCORES
CHUNK = 128
NEG = -3.0e38


def _lrelu(x):
    return jnp.where(x >= 0, x, 0.2 * x)


def _make_segsum(n_pad, d, n_chunks):
    stripe = n_pad // N_SUBCORES
    n_stage = stripe // CHUNK

    mesh = plsc.VectorSubcoreMesh(core_axis_name="c", subcore_axis_name="s")

    assert n_chunks % 2 == 0

    def body(u_hbm, src_hbm, dst_hbm, out_hbm, src_v, dst_v, rows0, rows1,
             s_sh, sem0, sem1):
        c = lax.axis_index("c")
        s = lax.axis_index("s")
        wid = c * N_SUBCORES + s

        def zrow(i, carry):
            for k in range(d // 16):
                rows0[i, pl.ds(k * 16, 16)] = jnp.zeros((16,), jnp.float32)
            return carry
        lax.fori_loop(0, CHUNK, zrow, 0)

        for t in range(n_stage):
            r0 = s * stripe + t * CHUNK
            pltpu.sync_copy(rows0, s_sh.at[pl.ds(r0, CHUNK)])
        plsc.subcore_barrier()

        nh = n_chunks // 2
        for ph in range(2):
            base = wid * n_chunks + ph * nh
            pltpu.sync_copy(src_hbm.at[pl.ds(base, nh)], src_v)
            pltpu.sync_copy(dst_hbm.at[pl.ds(base, nh)], dst_v)

            pltpu.async_copy(u_hbm.at[src_v.at[0]], rows0, sem0)

            @pl.loop(0, nh - 2, step=2)
            def _(j):
                pltpu.async_copy(u_hbm.at[src_v.at[j + 1]], rows1, sem1)
                pltpu.make_async_copy(u_hbm.at[src_v.at[j]], rows0,
                                      sem0).wait()
                pltpu.sync_copy(rows0, s_sh.at[dst_v.at[j]], add=True)
                pltpu.async_copy(u_hbm.at[src_v.at[j + 2]], rows0, sem0)
                pltpu.make_async_copy(u_hbm.at[src_v.at[j + 1]], rows1,
                                      sem1).wait()
                pltpu.sync_copy(rows1, s_sh.at[dst_v.at[j + 1]], add=True)

            jl = nh - 2
            pltpu.async_copy(u_hbm.at[src_v.at[jl + 1]], rows1, sem1)
            pltpu.make_async_copy(u_hbm.at[src_v.at[jl]], rows0, sem0).wait()
            pltpu.sync_copy(rows0, s_sh.at[dst_v.at[jl]], add=True)
            pltpu.make_async_copy(u_hbm.at[src_v.at[jl + 1]], rows1,
                                  sem1).wait()
            pltpu.sync_copy(rows1, s_sh.at[dst_v.at[jl + 1]], add=True)
        plsc.subcore_barrier()

        for t in range(n_stage):
            r0 = s * stripe + t * CHUNK
            pltpu.sync_copy(s_sh.at[pl.ds(r0, CHUNK)], rows0)
            pltpu.sync_copy(rows0, out_hbm.at[pl.ds(c * n_pad + r0, CHUNK)])

    return pl.kernel(
        body,
        out_type=[jax.ShapeDtypeStruct((2 * n_pad, d), jnp.float32)],
        mesh=mesh,
        scratch_types=[
            pltpu.VMEM((n_chunks // 2, CHUNK), jnp.int32),
            pltpu.VMEM((n_chunks // 2, CHUNK), jnp.int32),
            pltpu.VMEM((CHUNK, d), jnp.float32),
            pltpu.VMEM((CHUNK, d), jnp.float32),
            pltpu.VMEM_SHARED((n_pad, d), jnp.float32),
            pltpu.SemaphoreType.DMA,
            pltpu.SemaphoreType.DMA,
        ])


def _make_deg(n_pad, n_chunks):
    stripe = n_pad // N_SUBCORES
    n_stage = stripe // CHUNK

    mesh = plsc.VectorSubcoreMesh(core_axis_name="c", subcore_axis_name="s")

    def body(dst_hbm, out_hbm, dst_v, ones_v, deg_sh, sem):
        c = lax.axis_index("c")
        s = lax.axis_index("s")
        wid = c * N_SUBCORES + s

        def zrow(i, carry):
            for k in range(128 // 16):
                ones_v[i, pl.ds(k * 16, 16)] = jnp.zeros((16,), jnp.float32)
            return carry
        lax.fori_loop(0, CHUNK, zrow, 0)
        for t in range(n_stage):
            r0 = s * stripe + t * CHUNK
            pltpu.sync_copy(ones_v, deg_sh.at[pl.ds(r0, CHUNK)])

        def orow(i, carry):
            for k in range(128 // 16):
                ones_v[i, pl.ds(k * 16, 16)] = jnp.ones((16,), jnp.float32)
            return carry
        lax.fori_loop(0, CHUNK, orow, 0)
        plsc.subcore_barrier()

        pltpu.sync_copy(dst_hbm.at[pl.ds(wid * n_chunks, n_chunks)], dst_v)

        def step(j, carry):
            pltpu.sync_copy(ones_v, deg_sh.at[dst_v.at[j]], add=True)
            return carry
        lax.fori_loop(0, n_chunks, step, 0)
        plsc.subcore_barrier()

        for t in range(n_stage):
            r0 = s * stripe + t * CHUNK
            pltpu.sync_copy(deg_sh.at[pl.ds(r0, CHUNK)], ones_v)
            pltpu.sync_copy(ones_v, out_hbm.at[pl.ds(c * n_pad + r0, CHUNK)])

    return pl.kernel(
        body,
        out_type=[jax.ShapeDtypeStruct((2 * n_pad, 128), jnp.float32)],
        mesh=mesh,
        scratch_types=[
            pltpu.VMEM((n_chunks, CHUNK), jnp.int32),
            pltpu.VMEM((CHUNK, 128), jnp.float32),
            pltpu.VMEM_SHARED((n_pad, 128), jnp.float32),
            pltpu.SemaphoreType.DMA,
        ])


_BM = 256


def _tc_mlp2(x, w1t, w2t, bias):
    n = x.shape[0]

    def body(x_ref, w1_ref, w2_ref, b_ref, o_ref):
        h = jnp.dot(x_ref[...], w1_ref[...],
                    preferred_element_type=jnp.float32) + b_ref[0, :][None, :]
        h = _lrelu(h)
        h = jnp.dot(h, w2_ref[...],
                    preferred_element_type=jnp.float32) + b_ref[1, :][None, :]
        o_ref[...] = _lrelu(h)

    return pl.pallas_call(
        body,
        grid=(n // _BM,),
        in_specs=[pl.BlockSpec((_BM, 128), lambda i: (i, 0)),
                  pl.BlockSpec((128, 128), lambda i: (0, 0)),
                  pl.BlockSpec((128, 128), lambda i: (0, 0)),
                  pl.BlockSpec((8, 128), lambda i: (0, 0))],
        out_specs=pl.BlockSpec((_BM, 128), lambda i: (i, 0)),
        out_shape=jax.ShapeDtypeStruct((n, 128), jnp.float32),
    )(x, w1t, w2t, bias)


def _tc_mid(s0, s1, d0, d1, wlt, w1t, w2t, bias):
    n = s0.shape[0]

    def body(s0r, s1r, d0r, d1r, wl, w1, w2, br, o_ref):
        ssum = s0r[...] + s1r[...]
        deg = d0r[:, 0:1] + d1r[:, 0:1]
        mean = ssum / jnp.maximum(deg, 1.0)
        h = jnp.dot(mean, wl[...],
                    preferred_element_type=jnp.float32) + br[0, :][None, :]
        h = _lrelu(h)
        h = jnp.dot(h, w1[...],
                    preferred_element_type=jnp.float32) + br[1, :][None, :]
        h = _lrelu(h)
        h = jnp.dot(h, w2[...],
                    preferred_element_type=jnp.float32) + br[2, :][None, :]
        o_ref[...] = _lrelu(h)

    return pl.pallas_call(
        body,
        grid=(n // _BM,),
        in_specs=[pl.BlockSpec((_BM, 128), lambda i: (i, 0)),
                  pl.BlockSpec((_BM, 128), lambda i: (i, 0)),
                  pl.BlockSpec((_BM, 128), lambda i: (i, 0)),
                  pl.BlockSpec((_BM, 128), lambda i: (i, 0)),
                  pl.BlockSpec((128, 128), lambda i: (0, 0)),
                  pl.BlockSpec((128, 128), lambda i: (0, 0)),
                  pl.BlockSpec((128, 128), lambda i: (0, 0)),
                  pl.BlockSpec((8, 128), lambda i: (0, 0))],
        out_specs=pl.BlockSpec((_BM, 128), lambda i: (i, 0)),
        out_shape=jax.ShapeDtypeStruct((n, 128), jnp.float32),
    )(s0, s1, d0, d1, wlt, w1t, w2t, bias)


def _tc_final(s0, s1, d0, d1, wlt, bias, n_real):
    n = s0.shape[0]

    def body(s0r, s1r, d0r, d1r, wl, br, o_ref):
        i = pl.program_id(0)
        ssum = s0r[...] + s1r[...]
        deg = d0r[:, 0:1] + d1r[:, 0:1]
        mean = ssum / jnp.maximum(deg, 1.0)
        h = jnp.dot(mean, wl[...],
                    preferred_element_type=jnp.float32) + br[0, :][None, :]
        rid = i * _BM + lax.broadcasted_iota(jnp.int32, (_BM, 1), 0)
        h = jnp.where(rid < n_real, h, NEG)
        bmax = jnp.broadcast_to(jnp.max(h, axis=0, keepdims=True), (8, 128))

        @pl.when(i == 0)
        def _():
            o_ref[...] = jnp.full((8, 128), NEG, jnp.float32)

        o_ref[...] = jnp.maximum(o_ref[...], bmax)

    return pl.pallas_call(
        body,
        grid=(n // _BM,),
        in_specs=[pl.BlockSpec((_BM, 128), lambda i: (i, 0)),
                  pl.BlockSpec((_BM, 128), lambda i: (i, 0)),
                  pl.BlockSpec((_BM, 128), lambda i: (i, 0)),
                  pl.BlockSpec((_BM, 128), lambda i: (i, 0)),
                  pl.BlockSpec((128, 128), lambda i: (0, 0)),
                  pl.BlockSpec((8, 128), lambda i: (0, 0))],
        out_specs=pl.BlockSpec((8, 128), lambda i: (0, 0)),
        out_shape=jax.ShapeDtypeStruct((8, 128), jnp.float32),
    )(s0, s1, d0, d1, wlt, bias)


def _tc_head(mx, w1t, w2t, w3t, bias, n_out):
    def body(m_ref, w1, w2, w3, br, o_ref):
        h = jnp.dot(m_ref[...], w1[...],
                    preferred_element_type=jnp.float32) + br[0, :][None, :]
        h = _lrelu(h)
        h = jnp.dot(h, w2[...],
                    preferred_element_type=jnp.float32) + br[1, :][None, :]
        h = _lrelu(h)
        z = jnp.dot(h, w3[...],
                    preferred_element_type=jnp.float32) + br[2, :][None, :]
        col = lax.broadcasted_iota(jnp.int32, (1, 128), 1)
        valid = col < n_out
        zm = jnp.where(valid, z, NEG)
        m = jnp.max(zm, axis=1, keepdims=True)
        e = jnp.where(valid, jnp.exp(z - m), 0.0)
        lse = jnp.log(jnp.sum(e, axis=1, keepdims=True))
        o_ref[...] = z - m - lse

    return pl.pallas_call(
        body,
        grid=(1,),
        in_specs=[pl.BlockSpec((8, 128), lambda i: (0, 0)),
                  pl.BlockSpec((128, 128), lambda i: (0, 0)),
                  pl.BlockSpec((128, 128), lambda i: (0, 0)),
                  pl.BlockSpec((128, 128), lambda i: (0, 0)),
                  pl.BlockSpec((8, 128), lambda i: (0, 0))],
        out_specs=pl.BlockSpec((8, 128), lambda i: (0, 0)),
        out_shape=jax.ShapeDtypeStruct((8, 128), jnp.float32),
    )(mx, w1t, w2t, w3t, bias)


def _fold(lin, bn):
    s = bn["gamma"] / jnp.sqrt(1.0 + EPS)
    return (lin["W"] * s[:, None]).T, lin["b"] * s + bn["beta"]


def _bias_table(*rows):
    b = jnp.zeros((8, 128), jnp.float32)
    for i, r in enumerate(rows):
        b = b.at[i, : r.shape[0]].set(r)
    return b


def _sc_segsum(u, src2d, dst2d, n_pad):
    (out,) = _make_segsum(n_pad, u.shape[1], src2d.shape[0] // NW)(
        u, src2d, dst2d)
    return out


def _sc_deg(dst2d, n_pad):
    (out,) = _make_deg(n_pad, dst2d.shape[0] // NW)(dst2d)
    return out


def kernel(x, params, edge_index):
    p = params
    n, d = x.shape
    e = edge_index.shape[1]

    n_pad = ((n + 1 + 2047) // 2048) * 2048
    eq = NW * CHUNK * 8
    e_pad = ((e + eq - 1) // eq) * eq

    xp = jnp.pad(x, ((0, n_pad - n), (0, 0)))
    pad = jnp.full((e_pad - e,), n, jnp.int32)
    src2d = jnp.concatenate([edge_index[0], pad]).reshape(e_pad // CHUNK,
                                                          CHUNK)
    dst2d = jnp.concatenate([edge_index[1], pad]).reshape(e_pad // CHUNK,
                                                          CHUNK)

    w1a, b1a = _fold(p["mp1"]["fc1"], p["mp1"]["bn1"])
    w2a, b2a = _fold(p["mp1"]["fc2"], p["mp1"]["bn2"])
    wl1, bl1 = _fold(p["mp1"]["lin"], p["post1"])
    w1b, b1b = _fold(p["mp2"]["fc1"], p["mp2"]["bn1"])
    w2b, b2b = _fold(p["mp2"]["fc2"], p["mp2"]["bn2"])
    wl2, bl2 = _fold(p["mp2"]["lin"], p["post2"])
    w1c, b1c = _fold(p["mp3"]["fc1"], p["mp3"]["bn1"])
    w2c, b2c = _fold(p["mp3"]["fc2"], p["mp3"]["bn2"])
    wl3 = p["mp3"]["lin"]["W"].T
    bl3 = p["mp3"]["lin"]["b"]
    wf1, bf1 = _fold(p["fc1"]["lin"], p["fc1"]["bn"])
    wf2, bf2 = _fold(p["fc2"]["lin"], p["fc2"]["bn"])
    n_out = p["fc_final"]["W"].shape[0]
    wf3 = jnp.zeros((128, 128), jnp.float32).at[:, :n_out].set(
        p["fc_final"]["W"].T)
    bf3 = p["fc_final"]["b"]

    deg_parts = _sc_deg(dst2d, n_pad)
    d0, d1 = deg_parts[:n_pad], deg_parts[n_pad:]

    u1 = _tc_mlp2(xp, w1a, w2a, _bias_table(b1a, b2a))
    s_parts = _sc_segsum(u1, src2d, dst2d, n_pad)
    u2 = _tc_mid(s_parts[:n_pad], s_parts[n_pad:], d0, d1,
                 wl1, w1b, w2b, _bias_table(bl1, b1b, b2b))
    s_parts = _sc_segsum(u2, src2d, dst2d, n_pad)
    u3 = _tc_mid(s_parts[:n_pad], s_parts[n_pad:], d0, d1,
                 wl2, w1c, w2c, _bias_table(bl2, b1c, b2c))
    s_parts = _sc_segsum(u3, src2d, dst2d, n_pad)
    mx = _tc_final(s_parts[:n_pad], s_parts[n_pad:], d0, d1,
                   wl3, _bias_table(bl3), n)
    out = _tc_head(mx, wf1, wf2, wf3, _bias_table(bf1, bf2, bf3), n_out)
    return out[0:1, 0:n_out]


# speedup vs baseline: 8.8593x; 2.5278x over previous
import functools

import jax
import jax.numpy as jnp
from jax import lax
from jax.experimental import pallas as pl
from jax.experimental.pallas import tpu as pltpu
from jax.experimental.pallas import tpu_sc as plsc

EPS = 1e-5
N_CORES = 2
N_SUBCORES = 16
NW = N_CORES * N_SUBCORES
CHUNK = 128
NEG = -3.0e38


def _lrelu(x):
    return jnp.where(x >= 0, x, 0.2 * x)


def _make_segsum(n_pad, d, n_chunks):
    stripe = n_pad // N_SUBCORES
    n_stage = stripe // CHUNK

    mesh = plsc.VectorSubcoreMesh(core_axis_name="c", subcore_axis_name="s")

    assert n_chunks % 2 == 0

    def body(u_hbm, src_hbm, dst_hbm, out_hbm, src_v, dst_v, rows0, rows1,
             s_sh, sem0, sem1):
        c = lax.axis_index("c")
        s = lax.axis_index("s")
        wid = c * N_SUBCORES + s

        def zrow(i, carry):
            for k in range(d // 16):
                rows0[i, pl.ds(k * 16, 16)] = jnp.zeros((16,), jnp.float32)
            return carry
        lax.fori_loop(0, CHUNK, zrow, 0)

        for t in range(n_stage):
            r0 = s * stripe + t * CHUNK
            pltpu.sync_copy(rows0, s_sh.at[pl.ds(r0, CHUNK)])
        plsc.subcore_barrier()

        nh = n_chunks // 2
        for ph in range(2):
            base = wid * n_chunks + ph * nh
            pltpu.sync_copy(src_hbm.at[pl.ds(base, nh)], src_v)
            pltpu.sync_copy(dst_hbm.at[pl.ds(base, nh)], dst_v)

            pltpu.async_copy(u_hbm.at[src_v.at[0]], rows0, sem0)

            @pl.loop(0, nh - 2, step=2)
            def _(j):
                pltpu.async_copy(u_hbm.at[src_v.at[j + 1]], rows1, sem1)
                pltpu.make_async_copy(u_hbm.at[src_v.at[j]], rows0,
                                      sem0).wait()
                pltpu.sync_copy(rows0, s_sh.at[dst_v.at[j]], add=True)
                pltpu.async_copy(u_hbm.at[src_v.at[j + 2]], rows0, sem0)
                pltpu.make_async_copy(u_hbm.at[src_v.at[j + 1]], rows1,
                                      sem1).wait()
                pltpu.sync_copy(rows1, s_sh.at[dst_v.at[j + 1]], add=True)

            jl = nh - 2
            pltpu.async_copy(u_hbm.at[src_v.at[jl + 1]], rows1, sem1)
            pltpu.make_async_copy(u_hbm.at[src_v.at[jl]], rows0, sem0).wait()
            pltpu.sync_copy(rows0, s_sh.at[dst_v.at[jl]], add=True)
            pltpu.make_async_copy(u_hbm.at[src_v.at[jl + 1]], rows1,
                                  sem1).wait()
            pltpu.sync_copy(rows1, s_sh.at[dst_v.at[jl + 1]], add=True)
        plsc.subcore_barrier()

        for t in range(n_stage):
            r0 = s * stripe + t * CHUNK
            pltpu.sync_copy(s_sh.at[pl.ds(r0, CHUNK)], rows0)
            pltpu.sync_copy(rows0, out_hbm.at[pl.ds(c * n_pad + r0, CHUNK)])

    return pl.kernel(
        body,
        out_type=[jax.ShapeDtypeStruct((2 * n_pad, d), jnp.float32)],
        mesh=mesh,
        scratch_types=[
            pltpu.VMEM((n_chunks // 2, CHUNK), jnp.int32),
            pltpu.VMEM((n_chunks // 2, CHUNK), jnp.int32),
            pltpu.VMEM((CHUNK, d), jnp.float32),
            pltpu.VMEM((CHUNK, d), jnp.float32),
            pltpu.VMEM_SHARED((n_pad, d), jnp.float32),
            pltpu.SemaphoreType.DMA,
            pltpu.SemaphoreType.DMA,
        ])


def _make_deg(n_pad, n_chunks):
    stripe = n_pad // N_SUBCORES
    n_stage = stripe // CHUNK

    mesh = plsc.VectorSubcoreMesh(core_axis_name="c", subcore_axis_name="s")

    def body(dst_hbm, out_hbm, dst_v, ones_v, deg_sh, sem):
        c = lax.axis_index("c")
        s = lax.axis_index("s")
        wid = c * N_SUBCORES + s

        def zrow(i, carry):
            for k in range(128 // 16):
                ones_v[i, pl.ds(k * 16, 16)] = jnp.zeros((16,), jnp.float32)
            return carry
        lax.fori_loop(0, CHUNK, zrow, 0)
        for t in range(n_stage):
            r0 = s * stripe + t * CHUNK
            pltpu.sync_copy(ones_v, deg_sh.at[pl.ds(r0, CHUNK)])

        def orow(i, carry):
            for k in range(128 // 16):
                ones_v[i, pl.ds(k * 16, 16)] = jnp.ones((16,), jnp.float32)
            return carry
        lax.fori_loop(0, CHUNK, orow, 0)
        plsc.subcore_barrier()

        pltpu.sync_copy(dst_hbm.at[pl.ds(wid * n_chunks, n_chunks)], dst_v)

        def step(j, carry):
            pltpu.sync_copy(ones_v, deg_sh.at[dst_v.at[j]], add=True)
            return carry
        lax.fori_loop(0, n_chunks, step, 0)
        plsc.subcore_barrier()

        for t in range(n_stage):
            r0 = s * stripe + t * CHUNK
            pltpu.sync_copy(deg_sh.at[pl.ds(r0, CHUNK)], ones_v)
            pltpu.sync_copy(ones_v, out_hbm.at[pl.ds(c * n_pad + r0, CHUNK)])

    return pl.kernel(
        body,
        out_type=[jax.ShapeDtypeStruct((2 * n_pad, 128), jnp.float32)],
        mesh=mesh,
        scratch_types=[
            pltpu.VMEM((n_chunks, CHUNK), jnp.int32),
            pltpu.VMEM((CHUNK, 128), jnp.float32),
            pltpu.VMEM_SHARED((n_pad, 128), jnp.float32),
            pltpu.SemaphoreType.DMA,
        ])


_BM = 256


def _tc_mlp2(x, w1t, w2t, bias):
    n = x.shape[0]

    def body(x_ref, w1_ref, w2_ref, b_ref, o_ref):
        h = jnp.dot(x_ref[...], w1_ref[...],
                    preferred_element_type=jnp.float32) + b_ref[0, :][None, :]
        h = _lrelu(h)
        h = jnp.dot(h, w2_ref[...],
                    preferred_element_type=jnp.float32) + b_ref[1, :][None, :]
        o_ref[...] = _lrelu(h)

    return pl.pallas_call(
        body,
        grid=(n // _BM,),
        in_specs=[pl.BlockSpec((_BM, 128), lambda i: (i, 0)),
                  pl.BlockSpec((128, 128), lambda i: (0, 0)),
                  pl.BlockSpec((128, 128), lambda i: (0, 0)),
                  pl.BlockSpec((8, 128), lambda i: (0, 0))],
        out_specs=pl.BlockSpec((_BM, 128), lambda i: (i, 0)),
        out_shape=jax.ShapeDtypeStruct((n, 128), jnp.float32),
    )(x, w1t, w2t, bias)


def _tc_mid(s0, s1, d0, d1, wlt, w1t, w2t, bias):
    n = s0.shape[0]

    def body(s0r, s1r, d0r, d1r, wl, w1, w2, br, o_ref):
        ssum = s0r[...] + s1r[...]
        deg = d0r[:, 0:1] + d1r[:, 0:1]
        mean = ssum / jnp.maximum(deg, 1.0)
        h = jnp.dot(mean, wl[...],
                    preferred_element_type=jnp.float32) + br[0, :][None, :]
        h = _lrelu(h)
        h = jnp.dot(h, w1[...],
                    preferred_element_type=jnp.float32) + br[1, :][None, :]
        h = _lrelu(h)
        h = jnp.dot(h, w2[...],
                    preferred_element_type=jnp.float32) + br[2, :][None, :]
        o_ref[...] = _lrelu(h)

    return pl.pallas_call(
        body,
        grid=(n // _BM,),
        in_specs=[pl.BlockSpec((_BM, 128), lambda i: (i, 0)),
                  pl.BlockSpec((_BM, 128), lambda i: (i, 0)),
                  pl.BlockSpec((_BM, 128), lambda i: (i, 0)),
                  pl.BlockSpec((_BM, 128), lambda i: (i, 0)),
                  pl.BlockSpec((128, 128), lambda i: (0, 0)),
                  pl.BlockSpec((128, 128), lambda i: (0, 0)),
                  pl.BlockSpec((128, 128), lambda i: (0, 0)),
                  pl.BlockSpec((8, 128), lambda i: (0, 0))],
        out_specs=pl.BlockSpec((_BM, 128), lambda i: (i, 0)),
        out_shape=jax.ShapeDtypeStruct((n, 128), jnp.float32),
    )(s0, s1, d0, d1, wlt, w1t, w2t, bias)


def _tc_final(s0, s1, d0, d1, wlt, bias, n_real):
    n = s0.shape[0]

    def body(s0r, s1r, d0r, d1r, wl, br, o_ref):
        i = pl.program_id(0)
        ssum = s0r[...] + s1r[...]
        deg = d0r[:, 0:1] + d1r[:, 0:1]
        mean = ssum / jnp.maximum(deg, 1.0)
        h = jnp.dot(mean, wl[...],
                    preferred_element_type=jnp.float32) + br[0, :][None, :]
        rid = i * _BM + lax.broadcasted_iota(jnp.int32, (_BM, 1), 0)
        h = jnp.where(rid < n_real, h, NEG)
        bmax = jnp.broadcast_to(jnp.max(h, axis=0, keepdims=True), (8, 128))

        @pl.when(i == 0)
        def _():
            o_ref[...] = jnp.full((8, 128), NEG, jnp.float32)

        o_ref[...] = jnp.maximum(o_ref[...], bmax)

    return pl.pallas_call(
        body,
        grid=(n // _BM,),
        in_specs=[pl.BlockSpec((_BM, 128), lambda i: (i, 0)),
                  pl.BlockSpec((_BM, 128), lambda i: (i, 0)),
                  pl.BlockSpec((_BM, 128), lambda i: (i, 0)),
                  pl.BlockSpec((_BM, 128), lambda i: (i, 0)),
                  pl.BlockSpec((128, 128), lambda i: (0, 0)),
                  pl.BlockSpec((8, 128), lambda i: (0, 0))],
        out_specs=pl.BlockSpec((8, 128), lambda i: (0, 0)),
        out_shape=jax.ShapeDtypeStruct((8, 128), jnp.float32),
    )(s0, s1, d0, d1, wlt, bias)


def _tc_head(mx, w1t, w2t, w3t, bias, n_out):
    def body(m_ref, w1, w2, w3, br, o_ref):
        h = jnp.dot(m_ref[...], w1[...],
                    preferred_element_type=jnp.float32) + br[0, :][None, :]
        h = _lrelu(h)
        h = jnp.dot(h, w2[...],
                    preferred_element_type=jnp.float32) + br[1, :][None, :]
        h = _lrelu(h)
        z = jnp.dot(h, w3[...],
                    preferred_element_type=jnp.float32) + br[2, :][None, :]
        col = lax.broadcasted_iota(jnp.int32, (1, 128), 1)
        valid = col < n_out
        zm = jnp.where(valid, z, NEG)
        m = jnp.max(zm, axis=1, keepdims=True)
        e = jnp.where(valid, jnp.exp(z - m), 0.0)
        lse = jnp.log(jnp.sum(e, axis=1, keepdims=True))
        o_ref[...] = z - m - lse

    return pl.pallas_call(
        body,
        grid=(1,),
        in_specs=[pl.BlockSpec((8, 128), lambda i: (0, 0)),
                  pl.BlockSpec((128, 128), lambda i: (0, 0)),
                  pl.BlockSpec((128, 128), lambda i: (0, 0)),
                  pl.BlockSpec((128, 128), lambda i: (0, 0)),
                  pl.BlockSpec((8, 128), lambda i: (0, 0))],
        out_specs=pl.BlockSpec((8, 128), lambda i: (0, 0)),
        out_shape=jax.ShapeDtypeStruct((8, 128), jnp.float32),
    )(mx, w1t, w2t, w3t, bias)


def _fold(lin, bn):
    s = bn["gamma"] / jnp.sqrt(1.0 + EPS)
    return (lin["W"] * s[:, None]).T, lin["b"] * s + bn["beta"]


def _bias_table(*rows):
    b = jnp.zeros((8, 128), jnp.float32)
    for i, r in enumerate(rows):
        b = b.at[i, : r.shape[0]].set(r)
    return b


def _sc_segsum(u, src2d, dst2d, n_pad):
    (out,) = _make_segsum(n_pad, u.shape[1], src2d.shape[0] // NW)(
        u, src2d, dst2d)
    return out


def _sc_deg(dst2d, n_pad):
    (out,) = _make_deg(n_pad, dst2d.shape[0] // NW)(dst2d)
    return out


def kernel(x, params, edge_index):
    p = params
    n, d = x.shape
    e = edge_index.shape[1]

    n_pad = ((n + 1 + 2047) // 2048) * 2048
    eq = NW * CHUNK * 8
    e_pad = ((e + eq - 1) // eq) * eq

    xp = jnp.pad(x, ((0, n_pad - n), (0, 0)))
    pad = n + jnp.arange(e_pad - e, dtype=jnp.int32) % (n_pad - n)
    src2d = jnp.concatenate([edge_index[0], pad]).reshape(e_pad // CHUNK,
                                                          CHUNK)
    dst2d = jnp.concatenate([edge_index[1], pad]).reshape(e_pad // CHUNK,
                                                          CHUNK)

    w1a, b1a = _fold(p["mp1"]["fc1"], p["mp1"]["bn1"])
    w2a, b2a = _fold(p["mp1"]["fc2"], p["mp1"]["bn2"])
    wl1, bl1 = _fold(p["mp1"]["lin"], p["post1"])
    w1b, b1b = _fold(p["mp2"]["fc1"], p["mp2"]["bn1"])
    w2b, b2b = _fold(p["mp2"]["fc2"], p["mp2"]["bn2"])
    wl2, bl2 = _fold(p["mp2"]["lin"], p["post2"])
    w1c, b1c = _fold(p["mp3"]["fc1"], p["mp3"]["bn1"])
    w2c, b2c = _fold(p["mp3"]["fc2"], p["mp3"]["bn2"])
    wl3 = p["mp3"]["lin"]["W"].T
    bl3 = p["mp3"]["lin"]["b"]
    wf1, bf1 = _fold(p["fc1"]["lin"], p["fc1"]["bn"])
    wf2, bf2 = _fold(p["fc2"]["lin"], p["fc2"]["bn"])
    n_out = p["fc_final"]["W"].shape[0]
    wf3 = jnp.zeros((128, 128), jnp.float32).at[:, :n_out].set(
        p["fc_final"]["W"].T)
    bf3 = p["fc_final"]["b"]

    deg_parts = _sc_deg(dst2d, n_pad)
    d0, d1 = deg_parts[:n_pad], deg_parts[n_pad:]

    u1 = _tc_mlp2(xp, w1a, w2a, _bias_table(b1a, b2a))
    s_parts = _sc_segsum(u1, src2d, dst2d, n_pad)
    u2 = _tc_mid(s_parts[:n_pad], s_parts[n_pad:], d0, d1,
                 wl1, w1b, w2b, _bias_table(bl1, b1b, b2b))
    s_parts = _sc_segsum(u2, src2d, dst2d, n_pad)
    u3 = _tc_mid(s_parts[:n_pad], s_parts[n_pad:], d0, d1,
                 wl2, w1c, w2c, _bias_table(bl2, b1c, b2c))
    s_parts = _sc_segsum(u3, src2d, dst2d, n_pad)
    mx = _tc_final(s_parts[:n_pad], s_parts[n_pad:], d0, d1,
                   wl3, _bias_table(bl3), n)
    out = _tc_head(mx, wf1, wf2, wf3, _bias_table(bf1, bf2, bf3), n_out)
    return out[0:1, 0:n_out]
